# baseline (speedup 1.0000x reference)
import jax
import jax.numpy as jnp
from jax.experimental import pallas as pl
from jax.experimental.pallas import tpu as pltpu


def _add_kernel(scale, x_ref, t_ref, o_ref):
    o_ref[...] = x_ref[...] + t_ref[...][None, :, :] * scale


def kernel(inputs, lookup_table):
    B, T, D = inputs.shape
    scale = float(D) ** 0.5
    Bb = 128
    out = pl.pallas_call(
        lambda x_ref, t_ref, o_ref: _add_kernel(scale, x_ref, t_ref, o_ref),
        grid=(B // Bb,),
        in_specs=[
            pl.BlockSpec((Bb, T, D), lambda i: (i, 0, 0)),
            pl.BlockSpec((T, D), lambda i: (0, 0)),
        ],
        out_specs=pl.BlockSpec((Bb, T, D), lambda i: (i, 0, 0)),
        out_shape=jax.ShapeDtypeStruct((B, T, D), jnp.float32),
        compiler_params=pltpu.CompilerParams(
            dimension_semantics=("parallel",),
        ),
    )(inputs, lookup_table)
    return out
